# baseline (speedup 1.0000x reference)
import functools

import jax
import jax.numpy as jnp
from jax import lax
from jax.experimental import pallas as pl
from jax.experimental.pallas import tpu as pltpu
from jax.experimental.pallas import tpu_sc as plsc

SOM_X, SOM_Y = 16, 16
N_CODES = SOM_X * SOM_Y
LATENT = 64
BATCH = 1024
TAB_ROWS = 264
_HI = lax.Precision.HIGHEST

_INFO = plsc.get_sparse_core_info()
_NC, _NS, _LN = _INFO.num_cores, _INFO.num_subcores, _INFO.num_lanes
_NW = _NC * _NS
_BPW = BATCH // _NW


def _lrelu(x):
    return jnp.where(x > 0, x, 0.01 * x)


def _rowb(ref):
    return ref[...][None, :]


def _enc_body(x_ref, eps_ref, e_ref, et_ref,
              w0_ref, b0_ref, w1_ref, b1_ref, wm_ref, bm_ref, wl_ref, bl_ref,
              wd_ref, bd_ref, wd0_ref, bd0_ref, wd1_ref, bd1_ref,
              wd2_ref, bd2_ref,
              ze_o, n_o, zq_o, tab_o, de_o, dq_o):
    x = x_ref[...]
    eps = eps_ref[...]
    e = e_ref[...]

    zc2 = jnp.zeros((N_CODES, LATENT), jnp.float32)
    zr2 = jnp.zeros((TAB_ROWS - N_CODES, 2 * LATENT), jnp.float32)
    tab_o[...] = jnp.concatenate(
        [jnp.concatenate([e, zc2], axis=1), zr2], axis=0)

    h = _lrelu(x * w0_ref[...] + b0_ref[...])
    h = _lrelu(jnp.dot(h, w1_ref[...]) + b1_ref[...])
    mu = jnp.dot(h, wm_ref[...]) + bm_ref[...]
    logvar = jnp.dot(h, wl_ref[...]) + bl_ref[...]
    z = mu + eps * jnp.exp(0.5 * logvar)
    ze_o[...] = z

    et = et_ref[...]
    eb2 = jnp.sum(et * et, axis=0, keepdims=True)
    s = eb2 - 2.0 * jnp.dot(z, et, precision=_HI)

    iota = lax.broadcasted_iota(jnp.int32, (BATCH, N_CODES), 1)
    big = jnp.float32(3.4e38)

    m1 = jnp.min(s, axis=1, keepdims=True)
    n1 = jnp.min(jnp.where(s == m1, iota, N_CODES * 2), axis=1)
    s2 = jnp.where(iota == n1[:, None], big, s)
    m2 = jnp.min(s2, axis=1, keepdims=True)
    n2 = jnp.min(jnp.where(s2 == m2, iota, N_CODES * 2), axis=1)

    def onehot(idx):
        return (iota == idx[:, None]).astype(jnp.float32)

    oh = jnp.concatenate([onehot(n1), onehot(n2)], axis=0)
    g = jnp.dot(oh, e, precision=_HI)
    e1 = g[:BATCH]
    e2 = g[BATCH:]
    d1 = jnp.sum((e1 - z) * (e1 - z), axis=1)
    d2 = jnp.sum((e2 - z) * (e2 - z), axis=1)
    take2 = (d2 < d1) | ((d2 == d1) & (n2 < n1))
    n = jnp.where(take2, n2, n1)
    n_o[...] = jnp.reshape(n, (BATCH // 128, 128))
    zq = jnp.where(take2[:, None], e2, e1)
    zq_o[...] = zq

    zc = jnp.concatenate([z, zq], axis=0)
    t = _lrelu(jnp.dot(zc, wd_ref[...]) + bd_ref[...])
    t = _lrelu(jnp.dot(t, wd0_ref[...]) + bd0_ref[...])
    t = _lrelu(jnp.dot(t, wd1_ref[...]) + bd1_ref[...])
    d = _lrelu(jnp.dot(t, wd2_ref[...]) + bd2_ref[...])
    de_o[...] = d[:BATCH]
    dq_o[...] = d[BATCH:]


def _dec_body(ze_ref, zq_ref, wd_ref, bd_ref, wd0_ref, bd0_ref,
              wd1_ref, bd1_ref, wd2_ref, bd2_ref, de_o, dq_o):
    zc = jnp.concatenate([ze_ref[...], zq_ref[...]], axis=0)
    t = _lrelu(jnp.dot(zc, wd_ref[...]) + bd_ref[...])
    t = _lrelu(jnp.dot(t, wd0_ref[...]) + bd0_ref[...])
    t = _lrelu(jnp.dot(t, wd1_ref[...]) + bd1_ref[...])
    d = _lrelu(jnp.dot(t, wd2_ref[...]) + bd2_ref[...])
    de_o[...] = d[:BATCH]
    dq_o[...] = d[BATCH:]


def _sc_gather_body(tab_hbm, n_hbm, out_hbm, qi, ib, gr, sem):
    wid = lax.axis_index("s") * _NC + lax.axis_index("c")
    base = wid * _BPW
    pltpu.sync_copy(n_hbm.at[pl.ds(base, _BPW)], qi)
    for i in range(_BPW // _LN):
        sl = pl.ds(i * _LN, _LN)
        v = qi[sl]
        ib[pl.ds(i * _LN, _LN)] = (
            jnp.where(v < N_CODES - SOM_Y, v + SOM_Y, N_CODES))
        ib[pl.ds(_BPW + i * _LN, _LN)] = (
            jnp.where(v >= SOM_Y, v - SOM_Y, N_CODES))
        ib[pl.ds(2 * _BPW + i * _LN, _LN)] = (
            jnp.where((v & (SOM_Y - 1)) > 0, v - 1, N_CODES))
    pltpu.async_copy(tab_hbm.at[ib], gr, sem).wait()
    pltpu.sync_copy(gr, out_hbm.at[pl.ds(wid * 3 * _BPW, 3 * _BPW)])


def kernel(x, eps, embeddings, W_enc0, b_enc0, W_enc1, b_enc1, W_mu, b_mu,
           W_lv, b_lv, W_dec, b_dec, W_dec0, b_dec0, W_dec1, b_dec1,
           W_dec2, b_dec2):
    f32 = jnp.float32
    e_flat = embeddings.reshape(N_CODES, LATENT)
    e_t = e_flat.T

    def row(b):
        return b.reshape(1, -1)

    z_e, n2d, z_q, tab, de, dq = pl.pallas_call(
        _enc_body,
        out_shape=[
            jax.ShapeDtypeStruct((BATCH, LATENT), f32),
            jax.ShapeDtypeStruct((BATCH // 128, 128), jnp.int32),
            jax.ShapeDtypeStruct((BATCH, LATENT), f32),
            jax.ShapeDtypeStruct((TAB_ROWS, 2 * LATENT), f32),
            jax.ShapeDtypeStruct((BATCH, 1), f32),
            jax.ShapeDtypeStruct((BATCH, 1), f32),
        ],
    )(x, eps, e_flat, e_t,
      row(W_enc0.T.reshape(-1)), row(b_enc0), W_enc1.T, row(b_enc1),
      W_mu.T, row(b_mu), W_lv.T, row(b_lv),
      W_dec.T, row(b_dec), W_dec0.T, row(b_dec0), W_dec1.T, row(b_dec1),
      W_dec2.T, row(b_dec2))

    sc_gather = functools.partial(
        pl.kernel,
        mesh=plsc.VectorSubcoreMesh(core_axis_name="c", subcore_axis_name="s"),
        out_type=[
            jax.ShapeDtypeStruct((3 * BATCH, 2 * LATENT), f32),
        ],
        scratch_types=[
            pltpu.VMEM((_BPW,), jnp.int32),
            pltpu.VMEM((3 * _BPW,), jnp.int32),
            pltpu.VMEM((3 * _BPW, 2 * LATENT), f32),
            pltpu.SemaphoreType.DMA,
        ],
    )(_sc_gather_body)
    nbr = sc_gather(tab, n2d.reshape(BATCH))[0]

    nbv = nbr.reshape(_NW, 3, _BPW, 2 * LATENT)[:, :, :, :LATENT]
    up = nbv[:, 0].reshape(BATCH, LATENT)
    dn = nbv[:, 1].reshape(BATCH, LATENT)
    lf = nbv[:, 2].reshape(BATCH, LATENT)
    z_q_neighbors = jnp.stack(
        [z_q, up, dn, jnp.zeros_like(z_q), lf], axis=1)
    return (z_e, z_q, z_q_neighbors, de, dq)


# speedup vs baseline: 1.1196x; 1.1196x over previous
import functools

import jax
import jax.numpy as jnp
from jax import lax
from jax.experimental import pallas as pl
from jax.experimental.pallas import tpu as pltpu
from jax.experimental.pallas import tpu_sc as plsc

SOM_X, SOM_Y = 16, 16
N_CODES = SOM_X * SOM_Y
LATENT = 64
BATCH = 1024
TAB_ROWS = 264
_HI = lax.Precision.HIGHEST

_INFO = plsc.get_sparse_core_info()
_NC, _NS, _LN = _INFO.num_cores, _INFO.num_subcores, _INFO.num_lanes
_NW = _NC * _NS
_BPW = BATCH // _NW


def _lrelu(x):
    return jnp.where(x > 0, x, 0.01 * x)


def _rowb(ref):
    return ref[...][None, :]


def _enc_body(x_ref, eps_ref, e_ref, et_ref,
              w0_ref, b0_ref, w1_ref, b1_ref, wm_ref, bm_ref, wl_ref, bl_ref,
              ze_o, n_o, zq_o, tab_o):
    x = x_ref[...]
    eps = eps_ref[...]
    e = e_ref[...]

    zc2 = jnp.zeros((N_CODES, LATENT), jnp.float32)
    zr2 = jnp.zeros((TAB_ROWS - N_CODES, 2 * LATENT), jnp.float32)
    tab_o[...] = jnp.concatenate(
        [jnp.concatenate([e, zc2], axis=1), zr2], axis=0)

    h = _lrelu(x * w0_ref[...] + b0_ref[...])
    h = _lrelu(jnp.dot(h, w1_ref[...]) + b1_ref[...])
    mu = jnp.dot(h, wm_ref[...]) + bm_ref[...]
    logvar = jnp.dot(h, wl_ref[...]) + bl_ref[...]
    z = mu + eps * jnp.exp(0.5 * logvar)
    ze_o[...] = z

    et = et_ref[...]
    eb2 = jnp.sum(et * et, axis=0, keepdims=True)
    s = eb2 - 2.0 * jnp.dot(z, et, precision=_HI)

    iota = lax.broadcasted_iota(jnp.int32, (BATCH, N_CODES), 1)
    big = jnp.float32(3.4e38)

    m1 = jnp.min(s, axis=1, keepdims=True)
    n1 = jnp.min(jnp.where(s == m1, iota, N_CODES * 2), axis=1)
    s2 = jnp.where(iota == n1[:, None], big, s)
    m2 = jnp.min(s2, axis=1, keepdims=True)
    n2 = jnp.min(jnp.where(s2 == m2, iota, N_CODES * 2), axis=1)

    def onehot(idx):
        return (iota == idx[:, None]).astype(jnp.float32)

    oh = jnp.concatenate([onehot(n1), onehot(n2)], axis=0)
    g = jnp.dot(oh, e, precision=_HI)
    e1 = g[:BATCH]
    e2 = g[BATCH:]
    d1 = jnp.sum((e1 - z) * (e1 - z), axis=1)
    d2 = jnp.sum((e2 - z) * (e2 - z), axis=1)
    take2 = (d2 < d1) | ((d2 == d1) & (n2 < n1))
    n = jnp.where(take2, n2, n1)
    n_o[...] = jnp.reshape(n, (BATCH // 128, 128))
    zq_o[...] = jnp.where(take2[:, None], e2, e1)


def _dec_body(ze_ref, zq_ref, wd_ref, bd_ref, wd0_ref, bd0_ref,
              wd1_ref, bd1_ref, wd2_ref, bd2_ref, de_o, dq_o):
    zc = jnp.concatenate([ze_ref[...], zq_ref[...]], axis=0)
    t = _lrelu(jnp.dot(zc, wd_ref[...]) + bd_ref[...])
    t = _lrelu(jnp.dot(t, wd0_ref[...]) + bd0_ref[...])
    t = _lrelu(jnp.dot(t, wd1_ref[...]) + bd1_ref[...])
    d = _lrelu(jnp.dot(t, wd2_ref[...]) + bd2_ref[...])
    de_o[...] = d[:BATCH]
    dq_o[...] = d[BATCH:]


def _sc_gather_body(tab_hbm, n_hbm, out_hbm, qi, ib, gr, sem):
    wid = lax.axis_index("s") * _NC + lax.axis_index("c")
    base = wid * _BPW
    pltpu.sync_copy(n_hbm.at[pl.ds(base, _BPW)], qi)
    for i in range(_BPW // _LN):
        sl = pl.ds(i * _LN, _LN)
        v = qi[sl]
        ib[pl.ds(i * _LN, _LN)] = (
            jnp.where(v < N_CODES - SOM_Y, v + SOM_Y, N_CODES))
        ib[pl.ds(_BPW + i * _LN, _LN)] = (
            jnp.where(v >= SOM_Y, v - SOM_Y, N_CODES))
        ib[pl.ds(2 * _BPW + i * _LN, _LN)] = (
            jnp.where((v & (SOM_Y - 1)) > 0, v - 1, N_CODES))
    pltpu.async_copy(tab_hbm.at[ib], gr, sem).wait()
    pltpu.sync_copy(gr, out_hbm.at[pl.ds(wid * 3 * _BPW, 3 * _BPW)])


def kernel(x, eps, embeddings, W_enc0, b_enc0, W_enc1, b_enc1, W_mu, b_mu,
           W_lv, b_lv, W_dec, b_dec, W_dec0, b_dec0, W_dec1, b_dec1,
           W_dec2, b_dec2):
    f32 = jnp.float32
    e_flat = embeddings.reshape(N_CODES, LATENT)
    e_t = e_flat.T

    def row(b):
        return b.reshape(1, -1)

    z_e, n2d, z_q, tab = pl.pallas_call(
        _enc_body,
        out_shape=[
            jax.ShapeDtypeStruct((BATCH, LATENT), f32),
            jax.ShapeDtypeStruct((BATCH // 128, 128), jnp.int32),
            jax.ShapeDtypeStruct((BATCH, LATENT), f32),
            jax.ShapeDtypeStruct((TAB_ROWS, 2 * LATENT), f32),
        ],
    )(x, eps, e_flat, e_t,
      row(W_enc0.T.reshape(-1)), row(b_enc0), W_enc1.T, row(b_enc1),
      W_mu.T, row(b_mu), W_lv.T, row(b_lv))

    sc_gather = functools.partial(
        pl.kernel,
        mesh=plsc.VectorSubcoreMesh(core_axis_name="c", subcore_axis_name="s"),
        out_type=[
            jax.ShapeDtypeStruct((3 * BATCH, 2 * LATENT), f32),
        ],
        scratch_types=[
            pltpu.VMEM((_BPW,), jnp.int32),
            pltpu.VMEM((3 * _BPW,), jnp.int32),
            pltpu.VMEM((3 * _BPW, 2 * LATENT), f32),
            pltpu.SemaphoreType.DMA,
        ],
    )(_sc_gather_body)
    nbr = sc_gather(tab, n2d.reshape(BATCH))[0]

    de, dq = pl.pallas_call(
        _dec_body,
        out_shape=[jax.ShapeDtypeStruct((BATCH, 1), f32),
                   jax.ShapeDtypeStruct((BATCH, 1), f32)],
    )(z_e, z_q, W_dec.T, row(b_dec), W_dec0.T, row(b_dec0), W_dec1.T,
      row(b_dec1), W_dec2.T, row(b_dec2))

    nbv = nbr.reshape(_NW, 3, _BPW, 2 * LATENT)[:, :, :, :LATENT]
    up = nbv[:, 0].reshape(BATCH, LATENT)
    dn = nbv[:, 1].reshape(BATCH, LATENT)
    lf = nbv[:, 2].reshape(BATCH, LATENT)
    z_q_neighbors = jnp.stack(
        [z_q, up, dn, jnp.zeros_like(z_q), lf], axis=1)
    return (z_e, z_q, z_q_neighbors, de, dq)
